# baseline (speedup 1.0000x reference)
import jax
import jax.numpy as jnp
from jax.experimental import pallas as pl
from jax.experimental.pallas import tpu as pltpu

NUM_BAGS = 16
BM = 2048


def _fused_kernel(ids_ref, x_ref, w1_ref, b1_ref, w2_ref, b2_ref,
                  out_ref, sums_ref, counts_ref):
    i = pl.program_id(0)
    nb = pl.num_programs(0)
    h = jnp.dot(x_ref[...], w1_ref[...],
                preferred_element_type=jnp.float32)
    h = jnp.maximum(h + b1_ref[...], 0.0)
    ids = ids_ref[0]
    onehot = (jax.lax.broadcasted_iota(jnp.int32, (NUM_BAGS, BM), 0)
              == ids).astype(jnp.float32)
    part = jnp.dot(onehot, h, preferred_element_type=jnp.float32)
    cnt = jnp.broadcast_to(jnp.sum(onehot, axis=1, keepdims=True),
                           counts_ref.shape)

    @pl.when(i == 0)
    def _init():
        sums_ref[...] = part
        counts_ref[...] = cnt

    @pl.when(i != 0)
    def _acc():
        sums_ref[...] += part
        counts_ref[...] += cnt

    out_ref[...] = jnp.broadcast_to(b2_ref[...], out_ref.shape)

    @pl.when(i == nb - 1)
    def _top():
        means = sums_ref[...] / jnp.maximum(counts_ref[:, 0:1], 1.0)
        top = jnp.dot(means, w2_ref[...], preferred_element_type=jnp.float32)
        out_ref[0:NUM_BAGS, :] = top + b2_ref[...]


def kernel(x, ids, W1, b1, W2, b2):
    n, d = x.shape
    d_out = W2.shape[1]
    nb = n // BM
    ids3 = ids.reshape(nb, 1, BM)
    b1r = b1.reshape(1, d)
    b2r = b2.reshape(1, d_out)
    w1b = W1

    out = pl.pallas_call(
        _fused_kernel,
        grid=(nb,),
        in_specs=[
            pl.BlockSpec((1, 1, BM), lambda i: (i, 0, 0)),
            pl.BlockSpec((BM, d), lambda i: (i, 0)),
            pl.BlockSpec((d, d), lambda i: (0, 0)),
            pl.BlockSpec((1, d), lambda i: (0, 0)),
            pl.BlockSpec((d, d_out), lambda i: (0, 0)),
            pl.BlockSpec((1, d_out), lambda i: (0, 0)),
        ],
        out_specs=pl.BlockSpec((BM, d_out), lambda i: (pl.num_programs(0) - 1 - i, 0)),
        out_shape=jax.ShapeDtypeStruct((n, d_out), jnp.float32),
        scratch_shapes=[
            pltpu.VMEM((NUM_BAGS, d), jnp.float32),
            pltpu.VMEM((NUM_BAGS, 128), jnp.float32),
        ],
    )(ids3, x, w1b, b1r, W2, b2r)
    return out


# speedup vs baseline: 1.0737x; 1.0737x over previous
import jax
import jax.numpy as jnp
from jax.experimental import pallas as pl
from jax.experimental.pallas import tpu as pltpu

NUM_BAGS = 16
BM = 8192


def _fused_kernel(ids_ref, x_ref, w1_ref, b1_ref, w2_ref, b2_ref,
                  out_ref, sums_ref, counts_ref):
    i = pl.program_id(0)
    nb = pl.num_programs(0)
    h = jnp.dot(x_ref[...], w1_ref[...],
                preferred_element_type=jnp.float32)
    h = jnp.maximum(h + b1_ref[...], 0.0)
    ids = ids_ref[0]
    onehot = (jax.lax.broadcasted_iota(jnp.int32, (NUM_BAGS, BM), 0)
              == ids).astype(jnp.float32)
    part = jnp.dot(onehot, h, preferred_element_type=jnp.float32)
    cnt = jnp.broadcast_to(jnp.sum(onehot, axis=1, keepdims=True),
                           counts_ref.shape)

    @pl.when(i == 0)
    def _init():
        sums_ref[...] = part
        counts_ref[...] = cnt

    @pl.when(i != 0)
    def _acc():
        sums_ref[...] += part
        counts_ref[...] += cnt

    out_ref[...] = jnp.broadcast_to(b2_ref[...], out_ref.shape)

    @pl.when(i == nb - 1)
    def _top():
        means = sums_ref[...] / jnp.maximum(counts_ref[:, 0:1], 1.0)
        top = jnp.dot(means, w2_ref[...], preferred_element_type=jnp.float32)
        out_ref[0:NUM_BAGS, :] = top + b2_ref[...]


def kernel(x, ids, W1, b1, W2, b2):
    n, d = x.shape
    d_out = W2.shape[1]
    nb = n // BM
    ids3 = ids.reshape(nb, 1, BM)
    b1r = b1.reshape(1, d)
    b2r = b2.reshape(1, d_out)
    w1b = W1

    out = pl.pallas_call(
        _fused_kernel,
        grid=(nb,),
        in_specs=[
            pl.BlockSpec((1, 1, BM), lambda i: (i, 0, 0)),
            pl.BlockSpec((BM, d), lambda i: (i, 0)),
            pl.BlockSpec((d, d), lambda i: (0, 0)),
            pl.BlockSpec((1, d), lambda i: (0, 0)),
            pl.BlockSpec((d, d_out), lambda i: (0, 0)),
            pl.BlockSpec((1, d_out), lambda i: (0, 0)),
        ],
        out_specs=pl.BlockSpec((BM, d_out), lambda i: (pl.num_programs(0) - 1 - i, 0)),
        out_shape=jax.ShapeDtypeStruct((n, d_out), jnp.float32),
        compiler_params=pltpu.CompilerParams(
            vmem_limit_bytes=100 * 1024 * 1024),
        scratch_shapes=[
            pltpu.VMEM((NUM_BAGS, d), jnp.float32),
            pltpu.VMEM((NUM_BAGS, 128), jnp.float32),
        ],
    )(ids3, x, w1b, b1r, W2, b2r)
    return out


# speedup vs baseline: 1.0925x; 1.0175x over previous
import jax
import jax.numpy as jnp
from jax.experimental import pallas as pl
from jax.experimental.pallas import tpu as pltpu

NUM_BAGS = 16
BM = 4096


def _fused_kernel(ids_ref, x_ref, w1_ref, b1_ref, w2_ref, b2_ref,
                  out_ref, sums_ref, counts_ref):
    i = pl.program_id(0)
    nb = pl.num_programs(0)
    h = jnp.dot(x_ref[...], w1_ref[...],
                preferred_element_type=jnp.float32)
    h = jnp.maximum(h + b1_ref[...], 0.0)
    ids = ids_ref[0]
    onehot = (jax.lax.broadcasted_iota(jnp.int32, (NUM_BAGS, BM), 0)
              == ids).astype(jnp.float32)
    part = jnp.dot(onehot, h, preferred_element_type=jnp.float32)
    cnt = jnp.broadcast_to(jnp.sum(onehot, axis=1, keepdims=True),
                           counts_ref.shape)

    @pl.when(i == 0)
    def _init():
        sums_ref[...] = part
        counts_ref[...] = cnt

    @pl.when(i != 0)
    def _acc():
        sums_ref[...] += part
        counts_ref[...] += cnt

    out_ref[...] = jnp.broadcast_to(b2_ref[...], out_ref.shape)

    @pl.when(i == nb - 1)
    def _top():
        means = sums_ref[...] / jnp.maximum(counts_ref[:, 0:1], 1.0)
        top = jnp.dot(means, w2_ref[...], preferred_element_type=jnp.float32)
        out_ref[0:NUM_BAGS, :] = top + b2_ref[...]


def kernel(x, ids, W1, b1, W2, b2):
    n, d = x.shape
    d_out = W2.shape[1]
    nb = n // BM
    ids3 = ids.reshape(nb, 1, BM)
    b1r = b1.reshape(1, d)
    b2r = b2.reshape(1, d_out)
    w1b = W1

    out = pl.pallas_call(
        _fused_kernel,
        grid=(nb,),
        in_specs=[
            pl.BlockSpec((1, 1, BM), lambda i: (i, 0, 0)),
            pl.BlockSpec((BM, d), lambda i: (i, 0)),
            pl.BlockSpec((d, d), lambda i: (0, 0)),
            pl.BlockSpec((1, d), lambda i: (0, 0)),
            pl.BlockSpec((d, d_out), lambda i: (0, 0)),
            pl.BlockSpec((1, d_out), lambda i: (0, 0)),
        ],
        out_specs=pl.BlockSpec((BM, d_out), lambda i: (pl.num_programs(0) - 1 - i, 0)),
        out_shape=jax.ShapeDtypeStruct((n, d_out), jnp.float32),
        scratch_shapes=[
            pltpu.VMEM((NUM_BAGS, d), jnp.float32),
            pltpu.VMEM((NUM_BAGS, 128), jnp.float32),
        ],
    )(ids3, x, w1b, b1r, W2, b2r)
    return out
